# baseline (speedup 1.0000x reference)
import functools

import jax
import jax.numpy as jnp
from jax import lax
from jax.experimental import pallas as pl
from jax.experimental.pallas import tpu as pltpu
from jax.experimental.pallas import tpu_sc as plsc

N = 10000
E = 160000
D = 256
DH = 128
NC = 2
NS = 16
K = 80
EPW = E // NS
NCHUNK = EPW // K
RPS = 632
RLAST = N - (NS - 1) * RPS
CROW = 80

_sc_mesh = plsc.VectorSubcoreMesh(
    core_axis_name="c", subcore_axis_name="s", num_cores=NC, num_subcores=NS
)


@functools.partial(
    pl.kernel,
    out_type=(
        jax.ShapeDtypeStruct((N, DH), jnp.float32),
        jax.ShapeDtypeStruct((N, DH), jnp.float32),
        jax.ShapeDtypeStruct((NS, CROW, DH), jnp.float32),
    ),
    mesh=_sc_mesh,
    scratch_types=[
        pltpu.VMEM((NCHUNK, K), jnp.int32),
        pltpu.VMEM((NCHUNK, K), jnp.int32),
        pltpu.VMEM((K, DH), jnp.float32),
        pltpu.VMEM((K, DH), jnp.float32),
        pltpu.VMEM((CROW, DH), jnp.float32),
        pltpu.VMEM_SHARED((N, DH), jnp.float32),
        [pltpu.SemaphoreType.DMA] * 2,
        [pltpu.SemaphoreType.DMA] * 2,
    ],
    compiler_params=pltpu.CompilerParams(use_tc_tiling_on_sc=False, needs_layout_passes=False),
)
def _sc_aggregate(x0, x1, colsrc, rows2d, zeros,
                  outa, outb, cnts,
                  colv, rowv, g0, g1, cntloc, acc, sg, ss):
    c = lax.axis_index("c")
    s = lax.axis_index("s")

    @pl.when(s < NS - 1)
    def _():
        pltpu.sync_copy(zeros, acc.at[pl.ds(s * RPS, RPS)])

    @pl.when(s == NS - 1)
    def _():
        pltpu.sync_copy(zeros.at[pl.ds(0, RLAST)],
                        acc.at[pl.ds((NS - 1) * RPS, RLAST)])

    pltpu.sync_copy(zeros.at[pl.ds(0, CROW)], cntloc)
    pltpu.sync_copy(colsrc.at[s], colv)
    pltpu.sync_copy(rows2d.at[s], rowv)
    plsc.subcore_barrier()

    ones16 = jnp.full((16,), 1.0, jnp.float32)
    bufs = (g0, g1)

    def fire_g(i, b):
        @pl.when(c == 0)
        def _():
            pltpu.async_copy(x0.at[colv.at[i]], bufs[b], sg[b])

        @pl.when(c == 1)
        def _():
            pltpu.async_copy(x1.at[colv.at[i]], bufs[b], sg[b])

    def wait_g(i, b):
        pltpu.make_async_copy(x0.at[colv.at[i]], bufs[b], sg[b]).wait()

    def fire_s(i, b):
        pltpu.async_copy(bufs[b], acc.at[rowv.at[i]], ss[b], add=True)

    def wait_s(i, b):
        pltpu.make_async_copy(bufs[b], acc.at[rowv.at[i]], ss[b]).wait()

    def count(i):
        for j in range(K // 16):
            r16 = rowv[i, pl.ds(j * 16, 16)]
            hi = lax.shift_right_logical(r16, 7)
            lo = lax.bitwise_and(r16, 127)
            plsc.addupdate_scatter(cntloc, [hi, lo], ones16)

    fire_g(0, 0)
    fire_g(1, 1)

    def step(t, carry):
        i = 2 * t
        wait_g(i, 0)
        fire_s(i, 0)
        wait_g(i + 1, 1)
        fire_s(i + 1, 1)

        @pl.when(c == 0)
        def _():
            count(i)
            count(i + 1)
        wait_s(i, 0)
        fire_g(i + 2, 0)
        wait_s(i + 1, 1)

        @pl.when(i + 3 < NCHUNK)
        def _():
            fire_g(i + 3, 1)

        return carry

    lax.fori_loop(0, NCHUNK // 2, step, 0)
    i = NCHUNK - 1
    wait_g(i, 0)
    fire_s(i, 0)

    @pl.when(c == 0)
    def _():
        count(i)
    wait_s(i, 0)
    plsc.subcore_barrier()

    def write_out(dst):
        @pl.when(s < NS - 1)
        def _():
            sl = pl.ds(s * RPS, RPS)
            pltpu.sync_copy(acc.at[sl], dst.at[sl])

        @pl.when(s == NS - 1)
        def _():
            sl = pl.ds((NS - 1) * RPS, RLAST)
            pltpu.sync_copy(acc.at[sl], dst.at[sl])

    @pl.when(c == 0)
    def _():
        write_out(outa)
        pltpu.sync_copy(cntloc, cnts.at[s])

    @pl.when(c == 1)
    def _():
        write_out(outb)


R = 1024
NB = (N + R - 1) // R


def _tc_mlp(o0, o1, cb, w1t, b1, w2t, b2, out):
    inv = 1.0 / jnp.maximum(cb[...], 1.0)
    invc = jnp.transpose(inv, (1, 0))
    a0 = o0[...] * invc
    a1 = o1[...] * invc
    h = jnp.dot(a0, w1t[:DH, :], preferred_element_type=jnp.float32)
    h = h + jnp.dot(a1, w1t[DH:, :], preferred_element_type=jnp.float32)
    h = jnp.maximum(h + b1[...], 0.0)
    out[...] = jnp.dot(h, w2t[...], preferred_element_type=jnp.float32) + b2[...]


_tc_call = pl.pallas_call(
    _tc_mlp,
    grid=(NB,),
    in_specs=[
        pl.BlockSpec((R, DH), lambda i: (i, 0)),
        pl.BlockSpec((R, DH), lambda i: (i, 0)),
        pl.BlockSpec((1, R), lambda i: (0, i)),
        pl.BlockSpec((D, D), lambda i: (0, 0)),
        pl.BlockSpec((1, D), lambda i: (0, 0)),
        pl.BlockSpec((D, D), lambda i: (0, 0)),
        pl.BlockSpec((1, D), lambda i: (0, 0)),
    ],
    out_specs=pl.BlockSpec((R, D), lambda i: (i, 0)),
    out_shape=jax.ShapeDtypeStruct((N, D), jnp.float32),
)


def kernel(x, edge_index, W1, b1, W2, b2):
    row = edge_index[0].astype(jnp.int32)
    col = edge_index[1].astype(jnp.int32)

    x0 = x[:, :DH]
    x1 = x[:, DH:]
    colsrc = col.reshape(NS, NCHUNK, K)
    rows2d = row.reshape(NS, NCHUNK, K)
    zeros = jnp.zeros((RPS, DH), x.dtype)

    outa, outb, cnts = _sc_aggregate(x0, x1, colsrc, rows2d, zeros)
    cnt2 = cnts.sum(axis=0).reshape(1, CROW * DH)
    return _tc_call(outa, outb, cnt2,
                    W1.T, b1.reshape(1, D), W2.T, b2.reshape(1, D))


# speedup vs baseline: 1.1977x; 1.1977x over previous
import functools

import jax
import jax.numpy as jnp
from jax import lax
from jax.experimental import pallas as pl
from jax.experimental.pallas import tpu as pltpu
from jax.experimental.pallas import tpu_sc as plsc

N = 10000
E = 160000
D = 256
DH = 128
NC = 2
NS = 16
K = 80
EPW = E // NS
NCHUNK = EPW // K
RPS = 632
RLAST = N - (NS - 1) * RPS
CROW = 80

_sc_mesh = plsc.VectorSubcoreMesh(
    core_axis_name="c", subcore_axis_name="s", num_cores=NC, num_subcores=NS
)


@functools.partial(
    pl.kernel,
    out_type=(
        jax.ShapeDtypeStruct((N, DH), jnp.float32),
        jax.ShapeDtypeStruct((N, DH), jnp.float32),
        jax.ShapeDtypeStruct((NS, CROW, DH), jnp.float32),
    ),
    mesh=_sc_mesh,
    scratch_types=[
        pltpu.VMEM((NCHUNK, K), jnp.int32),
        pltpu.VMEM((NCHUNK, K), jnp.int32),
        pltpu.VMEM((K, DH), jnp.float32),
        pltpu.VMEM((K, DH), jnp.float32),
        pltpu.VMEM((CROW, DH), jnp.float32),
        pltpu.VMEM_SHARED((N, DH), jnp.float32),
        pltpu.SemaphoreType.DMA,
        pltpu.SemaphoreType.DMA,
        pltpu.SemaphoreType.DMA,
    ],
    compiler_params=pltpu.CompilerParams(use_tc_tiling_on_sc=False, needs_layout_passes=False),
)
def _sc_aggregate(x0, x1, colsrc, rows2d, zeros,
                  outa, outb, cnts,
                  colv, rowv, g0, g1, cntloc, acc, s0, s1, s2):
    c = lax.axis_index("c")
    s = lax.axis_index("s")

    @pl.when(s < NS - 1)
    def _():
        pltpu.sync_copy(zeros, acc.at[pl.ds(s * RPS, RPS)])

    @pl.when(s == NS - 1)
    def _():
        pltpu.sync_copy(zeros.at[pl.ds(0, RLAST)],
                        acc.at[pl.ds((NS - 1) * RPS, RLAST)])

    pltpu.sync_copy(zeros.at[pl.ds(0, CROW)], cntloc)
    pltpu.sync_copy(colsrc.at[s], colv)
    pltpu.sync_copy(rows2d.at[s], rowv)
    plsc.subcore_barrier()

    ones16 = jnp.full((16,), 1.0, jnp.float32)

    def fire(i, buf, sem, xtab):
        pltpu.async_copy(xtab.at[colv.at[i]], buf, sem)

    def fire2(i, buf, sem):
        @pl.when(c == 0)
        def _():
            fire(i, buf, sem, x0)

        @pl.when(c == 1)
        def _():
            fire(i, buf, sem, x1)

    def count(i):
        for j in range(K // 16):
            r16 = rowv[i, pl.ds(j * 16, 16)]
            hi = lax.shift_right_logical(r16, 7)
            lo = lax.bitwise_and(r16, 127)
            plsc.addupdate_scatter(cntloc, [hi, lo], ones16)

    def drain_scatter(i, buf, sem):
        pltpu.make_async_copy(x0.at[colv.at[i]], buf, sem).wait()
        pltpu.async_copy(buf, acc.at[rowv.at[i]], s2, add=True)

        @pl.when(c == 0)
        def _():
            count(i)
        pltpu.make_async_copy(buf, acc.at[rowv.at[i]], s2).wait()

    fire2(0, g0, s0)

    def step(i, carry):
        @pl.when(i % 2 == 0)
        def _():
            @pl.when(i + 1 < NCHUNK)
            def _():
                fire2(i + 1, g1, s1)
            drain_scatter(i, g0, s0)

        @pl.when(i % 2 == 1)
        def _():
            @pl.when(i + 1 < NCHUNK)
            def _():
                fire2(i + 1, g0, s0)
            drain_scatter(i, g1, s1)

        return carry

    lax.fori_loop(0, NCHUNK, step, 0)
    plsc.subcore_barrier()

    def write_out(dst):
        @pl.when(s < NS - 1)
        def _():
            sl = pl.ds(s * RPS, RPS)
            pltpu.sync_copy(acc.at[sl], dst.at[sl])

        @pl.when(s == NS - 1)
        def _():
            sl = pl.ds((NS - 1) * RPS, RLAST)
            pltpu.sync_copy(acc.at[sl], dst.at[sl])

    @pl.when(c == 0)
    def _():
        write_out(outa)
        pltpu.sync_copy(cntloc, cnts.at[s])

    @pl.when(c == 1)
    def _():
        write_out(outb)


R = 1024
NB = (N + R - 1) // R


def _tc_mlp(o0, o1, cb, w1t, b1, w2t, b2, out):
    inv = 1.0 / jnp.maximum(cb[...], 1.0)
    invc = jnp.transpose(inv, (1, 0))
    a0 = o0[...] * invc
    a1 = o1[...] * invc
    h = jnp.dot(a0, w1t[:DH, :], preferred_element_type=jnp.float32)
    h = h + jnp.dot(a1, w1t[DH:, :], preferred_element_type=jnp.float32)
    h = jnp.maximum(h + b1[...], 0.0)
    out[...] = jnp.dot(h, w2t[...], preferred_element_type=jnp.float32) + b2[...]


_tc_call = pl.pallas_call(
    _tc_mlp,
    grid=(NB,),
    in_specs=[
        pl.BlockSpec((R, DH), lambda i: (i, 0)),
        pl.BlockSpec((R, DH), lambda i: (i, 0)),
        pl.BlockSpec((1, R), lambda i: (0, i)),
        pl.BlockSpec((D, D), lambda i: (0, 0)),
        pl.BlockSpec((1, D), lambda i: (0, 0)),
        pl.BlockSpec((D, D), lambda i: (0, 0)),
        pl.BlockSpec((1, D), lambda i: (0, 0)),
    ],
    out_specs=pl.BlockSpec((R, D), lambda i: (i, 0)),
    out_shape=jax.ShapeDtypeStruct((N, D), jnp.float32),
)


def kernel(x, edge_index, W1, b1, W2, b2):
    row = edge_index[0].astype(jnp.int32)
    col = edge_index[1].astype(jnp.int32)

    x0 = x[:, :DH]
    x1 = x[:, DH:]
    colsrc = col.reshape(NS, NCHUNK, K)
    rows2d = row.reshape(NS, NCHUNK, K)
    zeros = jnp.zeros((RPS, DH), x.dtype)

    outa, outb, cnts = _sc_aggregate(x0, x1, colsrc, rows2d, zeros)
    cnt2 = cnts.sum(axis=0).reshape(1, CROW * DH)
    return _tc_call(outa, outb, cnt2,
                    W1.T, b1.reshape(1, D), W2.T, b2.reshape(1, D))
